# Initial kernel scaffold; baseline (speedup 1.0000x reference)
import jax
import jax.numpy as jnp
from jax.experimental import pallas as pl


def kernel(z, centers):
    raise NotImplementedError("write your pallas kernel here")


# speedup vs baseline: 1.4200x; 1.4200x over previous
import jax
import jax.numpy as jnp
from jax import lax
from jax.experimental import pallas as pl
from jax.experimental.pallas import tpu as pltpu

_TN = 1024


def _body(z_ref, c_ref, dist_ref, assign_ref):
    zb = z_ref[...]
    c = c_ref[...]
    dot = lax.dot_general(zb, c, (((1,), (1,)), ((), ())),
                          preferred_element_type=jnp.float32)
    z_sq = jnp.sum(zb * zb, axis=1, keepdims=True)
    c_sq = jnp.sum(c * c, axis=1)
    d2 = (z_sq - 2.0 * dot) + c_sq[None, :]
    a = jnp.argmin(d2, axis=1).astype(jnp.int32)
    onehot = lax.broadcasted_iota(jnp.int32, d2.shape, 1) == a[:, None]
    dot_a = jnp.sum(jnp.where(onehot, dot, 0.0), axis=1)
    c_sq_a = jnp.sum(jnp.where(onehot, c_sq[None, :], 0.0), axis=1)
    eps = 1e-8
    z_norm = jnp.maximum(jnp.sqrt(z_sq[:, 0]), eps)
    c_norm = jnp.maximum(jnp.sqrt(c_sq_a), eps)
    dist_ref[...] = 1.0 - dot_a / (z_norm * c_norm)
    assign_ref[...] = a


def kernel(z, centers):
    n, d = z.shape
    k, _ = centers.shape
    grid = (n // _TN,)
    dist, assign = pl.pallas_call(
        _body,
        grid=grid,
        in_specs=[
            pl.BlockSpec((_TN, d), lambda i: (i, 0)),
            pl.BlockSpec((k, d), lambda i: (0, 0)),
        ],
        out_specs=[
            pl.BlockSpec((_TN,), lambda i: (i,)),
            pl.BlockSpec((_TN,), lambda i: (i,)),
        ],
        out_shape=[
            jax.ShapeDtypeStruct((n,), jnp.float32),
            jax.ShapeDtypeStruct((n,), jnp.int32),
        ],
        compiler_params=pltpu.CompilerParams(
            dimension_semantics=("parallel",)),
    )(z, centers)
    return dist, assign


# speedup vs baseline: 2.0655x; 1.4546x over previous
import jax
import jax.numpy as jnp
from jax import lax
from jax.experimental import pallas as pl
from jax.experimental.pallas import tpu as pltpu

_TN = 1024
_EPS = 1e-8


def _body(z_ref, c_ref, dist_ref, assign_ref):
    zb = z_ref[...]
    c = c_ref[...]
    k, d = c.shape
    ones_col = jnp.ones((d, 1), jnp.float32)
    ones_row = jnp.ones((1, d), jnp.float32)
    cc = c * c
    c_sq_col = lax.dot_general(cc, ones_col, (((1,), (0,)), ((), ())),
                               preferred_element_type=jnp.float32)
    c_sq_row = lax.dot_general(ones_row, cc, (((1,), (1,)), ((), ())),
                               preferred_element_type=jnp.float32)
    z_sq = lax.dot_general(zb * zb, ones_col, (((1,), (0,)), ((), ())),
                           preferred_element_type=jnp.float32)
    z2 = jnp.concatenate([zb, jnp.ones((zb.shape[0], 1), jnp.float32)], 1)
    c2 = jnp.concatenate([-2.0 * c, c_sq_col], 1)
    g = lax.dot_general(z2, c2, (((1,), (1,)), ((), ())),
                        preferred_element_type=jnp.float32)
    m = jnp.min(g, axis=1, keepdims=True)
    eq = g == m
    idx_row = lax.broadcasted_iota(jnp.int32, (1, k), 1).astype(jnp.float32)
    a_f = jnp.min(jnp.where(eq, idx_row, jnp.float32(k)),
                  axis=1, keepdims=True)
    c_sq_a = jnp.max(jnp.where(eq, c_sq_row, -jnp.inf),
                     axis=1, keepdims=True)
    dot_a = 0.5 * (c_sq_a - m)
    inv_zn = lax.rsqrt(jnp.maximum(z_sq, _EPS * _EPS))
    inv_cn = lax.rsqrt(jnp.maximum(c_sq_a, _EPS * _EPS))
    dist_ref[...] = 1.0 - dot_a * (inv_zn * inv_cn)
    assign_ref[...] = a_f.astype(jnp.int32)


def kernel(z, centers):
    n, d = z.shape
    k, _ = centers.shape
    grid = (n // _TN,)
    dist, assign = pl.pallas_call(
        _body,
        grid=grid,
        in_specs=[
            pl.BlockSpec((_TN, d), lambda i: (i, 0)),
            pl.BlockSpec((k, d), lambda i: (0, 0)),
        ],
        out_specs=[
            pl.BlockSpec((_TN, 1), lambda i: (i, 0)),
            pl.BlockSpec((_TN, 1), lambda i: (i, 0)),
        ],
        out_shape=[
            jax.ShapeDtypeStruct((n, 1), jnp.float32),
            jax.ShapeDtypeStruct((n, 1), jnp.int32),
        ],
        compiler_params=pltpu.CompilerParams(
            dimension_semantics=("parallel",)),
    )(z, centers)
    return dist.reshape(n), assign.reshape(n)
